# baseline (speedup 1.0000x reference)
import functools

import jax
import jax.numpy as jnp
from jax import lax
from jax.experimental import pallas as pl
from jax.experimental.pallas import tpu as pltpu
from jax.experimental.pallas import tpu_sc as plsc

_VOCAB = 100000
_HIDDEN = 1024
_B = 4
_S = 4096
_EPS = 1e-12

_N_CHUNK = 4
_S_CHUNK = _S // _N_CHUNK
_N_ROWS = _B * _S_CHUNK
_NWORKERS = 32
_ROWS_PER_W = _N_ROWS // _NWORKERS
_W_PER_B = _S_CHUNK // _ROWS_PER_W
_GC = 32
_NSUB = _ROWS_PER_W // _GC

_R_BLK = 1024
_SBLK_PER_CHUNK = _S_CHUNK // _R_BLK
_SBLK_TOTAL = _B * _S // _R_BLK


_LANES = 16
_HALF = _HIDDEN // 2
_GROUPS = _HALF // _LANES


def _pack_rows(src_f32, dst_i32):
    def per_row(r, _):
        for j in range(_GROUPS):
            a = src_f32[r, pl.ds(j * _LANES, _LANES)]
            b = src_f32[r, pl.ds(_HALF + j * _LANES, _LANES)]
            p = plsc.pack(a, b, format=plsc.PackFormat.INTERLEAVED)
            dst_i32[r, pl.ds(j * _LANES, _LANES)] = plsc.bitcast(
                p, jnp.int32)
        return 0

    lax.fori_loop(0, _GC, per_row, 0)


def _make_gather_body(chunk):
    def body(tok_hbm, ids_hbm, out_hbm, idx_v, a0, a1, b0, b1,
             g0, g1, s0, s1):
        wid = lax.axis_index("s") * 2 + lax.axis_index("c")
        b = wid // _W_PER_B
        ids_base = b * _S + chunk * _S_CHUNK + (wid % _W_PER_B) * _ROWS_PER_W
        out_base = wid * _ROWS_PER_W
        pltpu.sync_copy(ids_hbm.at[pl.ds(ids_base, _ROWS_PER_W)], idx_v)

        abufs = (a0, a1)
        bbufs = (b0, b1)
        gsems = (g0, g1)
        ssems = (s0, s1)
        gathers = [None, None]
        scatters = [None, None]
        for g in range(_NSUB):
            if g == 0:
                gathers[0] = pltpu.async_copy(
                    tok_hbm.at[idx_v.at[pl.ds(0, _GC)]], abufs[0], gsems[0])
            if g >= 2:
                scatters[g % 2].wait()
            if g + 1 < _NSUB:
                gathers[(g + 1) % 2] = pltpu.async_copy(
                    tok_hbm.at[idx_v.at[pl.ds((g + 1) * _GC, _GC)]],
                    abufs[(g + 1) % 2], gsems[(g + 1) % 2])
            gathers[g % 2].wait()
            _pack_rows(abufs[g % 2], bbufs[g % 2])
            scatters[g % 2] = pltpu.async_copy(
                bbufs[g % 2], out_hbm.at[pl.ds(out_base + g * _GC, _GC)],
                ssems[g % 2])
        scatters[(_NSUB - 1) % 2].wait()
        scatters[(_NSUB - 2) % 2].wait()
    return body


def _make_sc_gather(chunk):
    return functools.partial(
        pl.kernel,
        out_type=jax.ShapeDtypeStruct((_N_ROWS, _HALF), jnp.int32),
        mesh=plsc.VectorSubcoreMesh(core_axis_name="c",
                                    subcore_axis_name="s"),
        compiler_params=pltpu.CompilerParams(needs_layout_passes=False),
        scratch_types=[
            pltpu.VMEM((_ROWS_PER_W,), jnp.int32),
            pltpu.VMEM((_GC, _HIDDEN), jnp.float32),
            pltpu.VMEM((_GC, _HIDDEN), jnp.float32),
            pltpu.VMEM((_GC, _HALF), jnp.int32),
            pltpu.VMEM((_GC, _HALF), jnp.int32),
            pltpu.SemaphoreType.DMA,
            pltpu.SemaphoreType.DMA,
            pltpu.SemaphoreType.DMA,
            pltpu.SemaphoreType.DMA,
        ],
    )(_make_gather_body(chunk))


_sc_gathers = [_make_sc_gather(c) for c in range(_N_CHUNK)]


def _ln_math(tok_ref, pos_ref, g_ref, b_ref, out_ref):
    v = tok_ref[...]
    lo = lax.bitcast_convert_type(lax.shift_left(v, 16), jnp.float32)
    hi = lax.bitcast_convert_type(
        lax.bitwise_and(v, jnp.int32(-65536)), jnp.float32)
    xl = lo + pos_ref[:, :_HALF]
    xh = hi + pos_ref[:, _HALF:]
    ssum = (jnp.sum(xl, axis=-1, keepdims=True)
            + jnp.sum(xh, axis=-1, keepdims=True))
    mean = ssum * (1.0 / _HIDDEN)
    xcl = xl - mean
    xch = xh - mean
    var = (jnp.sum(xcl * xcl, axis=-1, keepdims=True)
           + jnp.sum(xch * xch, axis=-1, keepdims=True)) * (1.0 / _HIDDEN)
    rstd = lax.rsqrt(var + _EPS)
    out_ref[:, :_HALF] = xcl * rstd * g_ref[:, :_HALF] + b_ref[:, :_HALF]
    out_ref[:, _HALF:] = xch * rstd * g_ref[:, _HALF:] + b_ref[:, _HALF:]


def _ln_body_acc(acc_ref, tok_ref, pos_ref, g_ref, b_ref, out_ref):
    del acc_ref
    _ln_math(tok_ref, pos_ref, g_ref, b_ref, out_ref)


def _make_tc_ln(chunk):
    pos0 = chunk * _SBLK_PER_CHUNK
    rows_spec = pl.BlockSpec((_R_BLK, _HALF),
                             lambda s, b: (b * _SBLK_PER_CHUNK + s, 0))
    pos_spec = pl.BlockSpec((_R_BLK, _HIDDEN),
                            lambda s, b: (pos0 + s, 0))
    vec_spec = pl.BlockSpec((1, _HIDDEN), lambda s, b: (0, 0))
    out_spec = pl.BlockSpec(
        (_R_BLK, _HIDDEN),
        lambda s, b: (b * (_S // _R_BLK) + pos0 + s, 0))
    out_shape = jax.ShapeDtypeStruct((_B * _S, _HIDDEN), jnp.float32)
    if chunk == 0:
        return pl.pallas_call(
            _ln_math,
            grid=(_SBLK_PER_CHUNK, _B),
            in_specs=[rows_spec, pos_spec, vec_spec, vec_spec],
            out_specs=out_spec,
            out_shape=out_shape,
        )
    return pl.pallas_call(
        _ln_body_acc,
        grid=(_SBLK_PER_CHUNK, _B),
        in_specs=[pl.BlockSpec(memory_space=pl.ANY),
                  rows_spec, pos_spec, vec_spec, vec_spec],
        out_specs=out_spec,
        out_shape=out_shape,
        input_output_aliases={0: 0},
    )


_tc_lns = [_make_tc_ln(c) for c in range(_N_CHUNK)]


def kernel(input_ids, token_table, pos_table, gamma, beta):
    g2 = gamma.reshape(1, _HIDDEN)
    b2 = beta.reshape(1, _HIDDEN)
    ids = input_ids.reshape(-1).astype(jnp.int32)
    rows = [_sc_gathers[c](token_table, ids) for c in range(_N_CHUNK)]
    acc = _tc_lns[0](rows[0], pos_table, g2, b2)
    for c in range(1, _N_CHUNK):
        acc = _tc_lns[c](acc, rows[c], pos_table, g2, b2)
    return acc.reshape(_B, _S, _HIDDEN)


# speedup vs baseline: 1.3251x; 1.3251x over previous
import functools

import jax
import jax.numpy as jnp
from jax import lax
from jax.experimental import pallas as pl
from jax.experimental.pallas import tpu as pltpu
from jax.experimental.pallas import tpu_sc as plsc

_VOCAB = 100000
_HIDDEN = 1024
_B = 4
_S = 4096
_EPS = 1e-12

_N_CHUNK = 4
_S_CHUNK = _S // _N_CHUNK
_N_ROWS = _B * _S_CHUNK
_NWORKERS = 32
_ROWS_PER_W = _N_ROWS // _NWORKERS
_W_PER_B = _S_CHUNK // _ROWS_PER_W
_GC = 32
_NSUB = _ROWS_PER_W // _GC

_R_BLK = 1024
_SBLK_PER_CHUNK = _S_CHUNK // _R_BLK
_SBLK_TOTAL = _B * _S // _R_BLK


_LANES = 16
_HALF = _HIDDEN // 2
_GROUPS = _HALF // _LANES


def _pack_rows(src_f32, dst_i32):
    @plsc.parallel_loop(0, _GC, unroll=2)
    def per_row(r):
        for j in range(_GROUPS):
            a = src_f32[r, pl.ds(j * _LANES, _LANES)]
            b = src_f32[r, pl.ds(_HALF + j * _LANES, _LANES)]
            p = plsc.pack(a, b, format=plsc.PackFormat.INTERLEAVED)
            dst_i32[r, pl.ds(j * _LANES, _LANES)] = plsc.bitcast(
                p, jnp.int32)


def _make_gather_body(chunk):
    def body(tok_hbm, ids_hbm, out_hbm, idx_v, a0, a1, b0, b1,
             g0, g1, s0, s1):
        wid = lax.axis_index("s") * 2 + lax.axis_index("c")
        b = wid // _W_PER_B
        ids_base = b * _S + chunk * _S_CHUNK + (wid % _W_PER_B) * _ROWS_PER_W
        out_base = wid * _ROWS_PER_W
        pltpu.sync_copy(ids_hbm.at[pl.ds(ids_base, _ROWS_PER_W)], idx_v)

        abufs = (a0, a1)
        bbufs = (b0, b1)
        gsems = (g0, g1)
        ssems = (s0, s1)
        gathers = [None, None]
        scatters = [None, None]
        for g in range(_NSUB):
            if g == 0:
                gathers[0] = pltpu.async_copy(
                    tok_hbm.at[idx_v.at[pl.ds(0, _GC)]], abufs[0], gsems[0])
            if g >= 2:
                scatters[g % 2].wait()
            if g + 1 < _NSUB:
                gathers[(g + 1) % 2] = pltpu.async_copy(
                    tok_hbm.at[idx_v.at[pl.ds((g + 1) * _GC, _GC)]],
                    abufs[(g + 1) % 2], gsems[(g + 1) % 2])
            gathers[g % 2].wait()
            _pack_rows(abufs[g % 2], bbufs[g % 2])
            scatters[g % 2] = pltpu.async_copy(
                bbufs[g % 2], out_hbm.at[pl.ds(out_base + g * _GC, _GC)],
                ssems[g % 2])
        scatters[(_NSUB - 1) % 2].wait()
        scatters[(_NSUB - 2) % 2].wait()
    return body


def _make_sc_gather(chunk):
    return functools.partial(
        pl.kernel,
        out_type=jax.ShapeDtypeStruct((_N_ROWS, _HALF), jnp.int32),
        mesh=plsc.VectorSubcoreMesh(core_axis_name="c",
                                    subcore_axis_name="s"),
        compiler_params=pltpu.CompilerParams(needs_layout_passes=False),
        scratch_types=[
            pltpu.VMEM((_ROWS_PER_W,), jnp.int32),
            pltpu.VMEM((_GC, _HIDDEN), jnp.float32),
            pltpu.VMEM((_GC, _HIDDEN), jnp.float32),
            pltpu.VMEM((_GC, _HALF), jnp.int32),
            pltpu.VMEM((_GC, _HALF), jnp.int32),
            pltpu.SemaphoreType.DMA,
            pltpu.SemaphoreType.DMA,
            pltpu.SemaphoreType.DMA,
            pltpu.SemaphoreType.DMA,
        ],
    )(_make_gather_body(chunk))


_sc_gathers = [_make_sc_gather(c) for c in range(_N_CHUNK)]


def _ln_math(tok_ref, pos_ref, g_ref, b_ref, out_ref):
    v = tok_ref[...]
    lo = lax.bitcast_convert_type(lax.shift_left(v, 16), jnp.float32)
    hi = lax.bitcast_convert_type(
        lax.bitwise_and(v, jnp.int32(-65536)), jnp.float32)
    xl = lo + pos_ref[:, :_HALF]
    xh = hi + pos_ref[:, _HALF:]
    ssum = (jnp.sum(xl, axis=-1, keepdims=True)
            + jnp.sum(xh, axis=-1, keepdims=True))
    mean = ssum * (1.0 / _HIDDEN)
    xcl = xl - mean
    xch = xh - mean
    var = (jnp.sum(xcl * xcl, axis=-1, keepdims=True)
           + jnp.sum(xch * xch, axis=-1, keepdims=True)) * (1.0 / _HIDDEN)
    rstd = lax.rsqrt(var + _EPS)
    out_ref[:, :_HALF] = xcl * rstd * g_ref[:, :_HALF] + b_ref[:, :_HALF]
    out_ref[:, _HALF:] = xch * rstd * g_ref[:, _HALF:] + b_ref[:, _HALF:]


def _ln_body_acc(acc_ref, tok_ref, pos_ref, g_ref, b_ref, out_ref):
    del acc_ref
    _ln_math(tok_ref, pos_ref, g_ref, b_ref, out_ref)


def _make_tc_ln(chunk):
    pos0 = chunk * _SBLK_PER_CHUNK
    rows_spec = pl.BlockSpec((_R_BLK, _HALF),
                             lambda s, b: (b * _SBLK_PER_CHUNK + s, 0))
    pos_spec = pl.BlockSpec((_R_BLK, _HIDDEN),
                            lambda s, b: (pos0 + s, 0))
    vec_spec = pl.BlockSpec((1, _HIDDEN), lambda s, b: (0, 0))
    out_spec = pl.BlockSpec(
        (_R_BLK, _HIDDEN),
        lambda s, b: (b * (_S // _R_BLK) + pos0 + s, 0))
    out_shape = jax.ShapeDtypeStruct((_B * _S, _HIDDEN), jnp.float32)
    if chunk == 0:
        return pl.pallas_call(
            _ln_math,
            grid=(_SBLK_PER_CHUNK, _B),
            in_specs=[rows_spec, pos_spec, vec_spec, vec_spec],
            out_specs=out_spec,
            out_shape=out_shape,
        )
    return pl.pallas_call(
        _ln_body_acc,
        grid=(_SBLK_PER_CHUNK, _B),
        in_specs=[pl.BlockSpec(memory_space=pl.ANY),
                  rows_spec, pos_spec, vec_spec, vec_spec],
        out_specs=out_spec,
        out_shape=out_shape,
        input_output_aliases={0: 0},
    )


_tc_lns = [_make_tc_ln(c) for c in range(_N_CHUNK)]


def kernel(input_ids, token_table, pos_table, gamma, beta):
    g2 = gamma.reshape(1, _HIDDEN)
    b2 = beta.reshape(1, _HIDDEN)
    ids = input_ids.reshape(-1).astype(jnp.int32)
    rows = [_sc_gathers[c](token_table, ids) for c in range(_N_CHUNK)]
    acc = _tc_lns[0](rows[0], pos_table, g2, b2)
    for c in range(1, _N_CHUNK):
        acc = _tc_lns[c](acc, rows[c], pos_table, g2, b2)
    return acc.reshape(_B, _S, _HIDDEN)
